# baseline (speedup 1.0000x reference)
import functools

import jax
import jax.numpy as jnp
from jax import lax
from jax.experimental import pallas as pl
from jax.experimental.pallas import tpu as pltpu
from jax.experimental.pallas import tpu_sc as plsc

B = 16384
D = 64
G = 8
NUM_GENRES = 32
NUM_TITLES_C = 1000000

NC = 2
NS = 16
NW = NC * NS
BPW = B // NW
CHUNK = 128
KCH = BPW // CHUNK

BLK = 1024


def _sc_gather_T(table_1d, idx3):
    mesh = plsc.VectorSubcoreMesh(
        core_axis_name="c", subcore_axis_name="s",
        num_cores=NC, num_subcores=NS)

    @functools.partial(
        pl.kernel,
        out_type=jax.ShapeDtypeStruct((D * B,), jnp.float32),
        mesh=mesh,
        scratch_types=[
            pltpu.VMEM((KCH, CHUNK), jnp.int32),
            pltpu.VMEM((KCH, CHUNK), jnp.int32),
            pltpu.VMEM((BPW * D,), jnp.float32),
            pltpu.SemaphoreType.DMA,
            pltpu.SemaphoreType.DMA,
        ],
        compiler_params=pltpu.CompilerParams(use_tc_tiling_on_sc=False),
    )
    def k(table_hbm, idx_hbm, out_hbm, idx_v, widx_v, cols_v, sem, wsem):
        wid = lax.axis_index("s") * NC + lax.axis_index("c")
        base = wid * BPW
        pltpu.sync_copy(idx_hbm.at[wid], idx_v)

        def per_feature(d, _):
            dbase = d * NUM_TITLES_C
            for c in range(KCH):
                for v in range(CHUNK // 16):
                    widx_v[c, pl.ds(v * 16, 16)] = (
                        idx_v[c, pl.ds(v * 16, 16)] + dbase)
            cps = []
            for c in range(KCH):
                cps.append(pltpu.async_copy(
                    table_hbm.at[widx_v.at[c]],
                    cols_v.at[pl.ds(d * BPW + c * CHUNK, CHUNK)],
                    sem))
            for cp in cps:
                cp.wait()
            return 0

        lax.fori_loop(0, D, per_feature, 0)

        def write_row(d, _):
            pltpu.async_copy(
                cols_v.at[pl.ds(d * BPW, BPW)],
                out_hbm.at[pl.ds(d * B + base, BPW)],
                wsem).wait()
            return 0

        lax.fori_loop(0, D, write_row, 0)

    return k(table_1d, idx3)


def _tc_body_T(gt_ref, w_ref, b_ref, titleT_ref, genresT_ref, outT_ref):
    gT = genresT_ref[...]
    cls = lax.broadcasted_iota(jnp.int32, (NUM_GENRES, 1), 0)
    counts = jnp.zeros((NUM_GENRES, BLK), jnp.float32)
    for j in range(G):
        counts += (gT[j:j + 1, :] == cls).astype(jnp.float32)
    genre_embT = lax.dot_general(
        gt_ref[...], counts, (((0,), (0,)), ((), ())),
        preferred_element_type=jnp.float32)
    combT = jnp.concatenate([titleT_ref[...], genre_embT], axis=0)
    outT = lax.dot_general(
        w_ref[...], combT, (((0,), (0,)), ((), ())),
        preferred_element_type=jnp.float32) + b_ref[...]
    outT_ref[...] = jnp.maximum(outT, 0.0)


def _tc_combine_T(titleT, genres_T, genre_table, W, b2):
    return pl.pallas_call(
        _tc_body_T,
        out_shape=jax.ShapeDtypeStruct((D, B), jnp.float32),
        grid=(B // BLK,),
        in_specs=[
            pl.BlockSpec((NUM_GENRES, D), lambda i: (0, 0)),
            pl.BlockSpec((2 * D, D), lambda i: (0, 0)),
            pl.BlockSpec((D, 1), lambda i: (0, 0)),
            pl.BlockSpec((D, BLK), lambda i: (0, i)),
            pl.BlockSpec((G, BLK), lambda i: (0, i)),
        ],
        out_specs=pl.BlockSpec((D, BLK), lambda i: (0, i)),
    )(genre_table, W, b2, titleT, genres_T)


def kernel(movie_title, movie_genres, title_table, genre_table, W, b):
    table_1d = title_table.T.reshape(-1)
    genres_T = movie_genres.T
    idx3 = movie_title.reshape(NW, KCH, CHUNK)
    titleT = _sc_gather_T(table_1d, idx3).reshape(D, B)
    outT = _tc_combine_T(titleT, genres_T, genre_table, W,
                         b.reshape(D, 1))
    return outT.T


# speedup vs baseline: 1.0073x; 1.0073x over previous
import functools

import jax
import jax.numpy as jnp
from jax import lax
from jax.experimental import pallas as pl
from jax.experimental.pallas import tpu as pltpu
from jax.experimental.pallas import tpu_sc as plsc

B = 16384
D = 64
G = 8
NUM_GENRES = 32
NUM_TITLES_C = 1000000

NC = 2
NS = 16
NW = NC * NS
BPW = B // NW
CHUNK = 128
KCH = BPW // CHUNK

BLK = 1024


PIPE = 4


def _sc_gather_T(table_T, idx3):
    mesh = plsc.VectorSubcoreMesh(
        core_axis_name="c", subcore_axis_name="s",
        num_cores=NC, num_subcores=NS)

    @functools.partial(
        pl.kernel,
        out_type=jax.ShapeDtypeStruct((D, B), jnp.float32),
        mesh=mesh,
        scratch_types=[
            pltpu.VMEM((KCH, CHUNK), jnp.int32),
            pltpu.VMEM((D, BPW), jnp.float32),
            pltpu.SemaphoreType.DMA,
        ],
        compiler_params=pltpu.CompilerParams(use_tc_tiling_on_sc=False),
    )
    def k(table_hbm, idx_hbm, out_hbm, idx_v, cols_v, sem):
        wid = lax.axis_index("s") * NC + lax.axis_index("c")
        base = wid * BPW
        pltpu.sync_copy(idx_hbm.at[wid], idx_v)

        def fire(d):
            for c in range(KCH):
                pltpu.async_copy(
                    table_hbm.at[d].at[idx_v.at[c]],
                    cols_v.at[d].at[pl.ds(c * CHUNK, CHUNK)],
                    sem)

        def drain(d):
            for c in range(KCH):
                pltpu.make_async_copy(
                    table_hbm.at[d].at[idx_v.at[c]],
                    cols_v.at[d].at[pl.ds(c * CHUNK, CHUNK)],
                    sem).wait()

        def steady(d, _):
            fire(d)
            drain(d - PIPE)
            return 0

        for d in range(PIPE):
            fire(d)
        lax.fori_loop(PIPE, D, steady, 0)

        def tail(d, _):
            drain(d)
            return 0

        lax.fori_loop(D - PIPE, D, tail, 0)

        pltpu.sync_copy(cols_v, out_hbm.at[:, pl.ds(base, BPW)])

    return k(table_T, idx3)


def _tc_body_T(gt_ref, w_ref, b_ref, titleT_ref, genresT_ref, outT_ref):
    gT = genresT_ref[...]
    cls = lax.broadcasted_iota(jnp.int32, (NUM_GENRES, 1), 0)
    counts = jnp.zeros((NUM_GENRES, BLK), jnp.float32)
    for j in range(G):
        counts += (gT[j:j + 1, :] == cls).astype(jnp.float32)
    genre_embT = lax.dot_general(
        gt_ref[...], counts, (((0,), (0,)), ((), ())),
        preferred_element_type=jnp.float32)
    combT = jnp.concatenate([titleT_ref[...], genre_embT], axis=0)
    outT = lax.dot_general(
        w_ref[...], combT, (((0,), (0,)), ((), ())),
        preferred_element_type=jnp.float32) + b_ref[...]
    outT_ref[...] = jnp.maximum(outT, 0.0)


def _tc_combine_T(titleT, genres_T, genre_table, W, b2):
    return pl.pallas_call(
        _tc_body_T,
        out_shape=jax.ShapeDtypeStruct((D, B), jnp.float32),
        grid=(B // BLK,),
        in_specs=[
            pl.BlockSpec((NUM_GENRES, D), lambda i: (0, 0)),
            pl.BlockSpec((2 * D, D), lambda i: (0, 0)),
            pl.BlockSpec((D, 1), lambda i: (0, 0)),
            pl.BlockSpec((D, BLK), lambda i: (0, i)),
            pl.BlockSpec((G, BLK), lambda i: (0, i)),
        ],
        out_specs=pl.BlockSpec((D, BLK), lambda i: (0, i)),
    )(genre_table, W, b2, titleT, genres_T)


def kernel(movie_title, movie_genres, title_table, genre_table, W, b):
    table_T = title_table.T
    genres_T = movie_genres.T
    idx3 = movie_title.reshape(NW, KCH, CHUNK)
    titleT = _sc_gather_T(table_T, idx3)
    outT = _tc_combine_T(titleT, genres_T, genre_table, W,
                         b.reshape(D, 1))
    return outT.T
